# baseline (speedup 1.0000x reference)
import jax
import jax.numpy as jnp
from jax import lax
from jax.experimental import pallas as pl
from jax.experimental.pallas import tpu as pltpu
from jax.experimental.pallas import tpu_sc as plsc

BATCH = 16384
EMBED_DIM = 64
CHUNK = 128
NBUF = 8

_info = plsc.get_sparse_core_info()
NUM_CORES = _info.num_cores
NUM_SUBCORES = _info.num_subcores
NUM_WORKERS = NUM_CORES * NUM_SUBCORES
ROWS_PER_WORKER = BATCH // NUM_WORKERS
SET_CHUNKS = ROWS_PER_WORKER // CHUNK
DBLOCKS = EMBED_DIM // NBUF


def _transe_kernel(entT, relT,
                   ph_hbm, pr_hbm, pt_hbm, nh_hbm, nr_hbm, nt_hbm,
                   pos_outT, neg_outT,
                   ihv, irv, itv, h_s, r_s, t_s, stripe,
                   sem0, sem1, sem2, sem3, sem4, sem5, sem6, sem7):
    wid = lax.axis_index("s") * NUM_CORES + lax.axis_index("c")
    wbase = wid * ROWS_PER_WORKER
    sems = [sem0, sem1, sem2, sem3, sem4, sem5, sem6, sem7]

    def fire(d, b):
        sl = pl.ds(b * CHUNK, CHUNK)
        pltpu.async_copy(entT.at[d].at[ihv], h_s.at[sl], sems[b])
        pltpu.async_copy(relT.at[d].at[irv], r_s.at[sl], sems[b])
        pltpu.async_copy(entT.at[d].at[itv], t_s.at[sl], sems[b])

    def drain(b):
        sl = pl.ds(b * CHUNK, CHUNK)
        pltpu.make_async_copy(entT.at[0].at[ihv], h_s.at[sl], sems[b]).wait()
        pltpu.make_async_copy(relT.at[0].at[irv], r_s.at[sl], sems[b]).wait()
        pltpu.make_async_copy(entT.at[0].at[itv], t_s.at[sl], sems[b]).wait()

    def run_set(ih_hbm, ir_hbm, it_hbm, outT):
        def chunk_body(c, carry):
            pltpu.sync_copy(ih_hbm.at[wid].at[c], ihv)
            pltpu.sync_copy(ir_hbm.at[wid].at[c], irv)
            pltpu.sync_copy(it_hbm.at[wid].at[c], itv)
            for b in range(NBUF):
                fire(b, b)

            def dblock_body(i, carry2):
                for b in range(NBUF):
                    d = i * NBUF + b
                    drain(b)
                    sb = b * CHUNK
                    for k in range(CHUNK // 16):
                        sl = pl.ds(sb + k * 16, 16)
                        so = pl.ds(k * 16, 16)
                        stripe[d, so] = jnp.abs(h_s[sl] + r_s[sl] - t_s[sl])

                    @pl.when(i < DBLOCKS - 1)
                    def _():
                        fire(d + NBUF, b)
                return carry2

            lax.fori_loop(0, DBLOCKS, dblock_body, 0)
            pltpu.sync_copy(
                stripe, outT.at[:, pl.ds(wbase + c * CHUNK, CHUNK)])
            return carry

        lax.fori_loop(0, SET_CHUNKS, chunk_body, 0)

    run_set(ph_hbm, pr_hbm, pt_hbm, pos_outT)
    run_set(nh_hbm, nr_hbm, nt_hbm, neg_outT)


@jax.jit
def kernel(positive_samples, negative_samples, entity_embedding, relation_embedding):
    idx_shape = (NUM_WORKERS, SET_CHUNKS, CHUNK)
    ph = positive_samples[:, 0].reshape(idx_shape)
    pr = positive_samples[:, 1].reshape(idx_shape)
    pt = positive_samples[:, 2].reshape(idx_shape)
    nh = negative_samples[:, 0].reshape(idx_shape)
    nr = negative_samples[:, 1].reshape(idx_shape)
    nt = negative_samples[:, 2].reshape(idx_shape)

    mesh = plsc.VectorSubcoreMesh(core_axis_name="c", subcore_axis_name="s")
    out_t = jax.ShapeDtypeStruct((EMBED_DIM, BATCH), jnp.float32)
    run = pl.kernel(
        _transe_kernel,
        out_type=(out_t, out_t),
        mesh=mesh,
        compiler_params=pltpu.CompilerParams(use_tc_tiling_on_sc=False),
        scratch_types=[
            pltpu.VMEM((CHUNK,), jnp.int32),
            pltpu.VMEM((CHUNK,), jnp.int32),
            pltpu.VMEM((CHUNK,), jnp.int32),
            pltpu.VMEM((NBUF * CHUNK,), jnp.float32),
            pltpu.VMEM((NBUF * CHUNK,), jnp.float32),
            pltpu.VMEM((NBUF * CHUNK,), jnp.float32),
            pltpu.VMEM((EMBED_DIM, CHUNK), jnp.float32),
        ] + [pltpu.SemaphoreType.DMA] * NBUF,
    )
    pos_T, neg_T = run(entity_embedding.T, relation_embedding.T,
                       ph, pr, pt, nh, nr, nt)
    return pos_T.T, neg_T.T
